# baseline (speedup 1.0000x reference)
import functools
import math

import jax
import jax.numpy as jnp
from jax import lax
from jax.experimental import pallas as pl
from jax.experimental.pallas import tpu as pltpu
from jax.experimental.pallas import tpu_sc as plsc

D_MODEL = 1024
SCALE = math.sqrt(D_MODEL)
LANES = 16
NW = 32
CH = 16
NBUF = 5
LOOKAHEAD = 3
SLOTS = 2


def kernel(x, table):
    B0, S = x.shape
    V, D = table.shape
    x = x.astype(jnp.int32)
    b_per_w = (B0 * S) // NW
    w_per_row = S // b_per_w
    n_ch = b_per_w // CH

    mesh = plsc.VectorSubcoreMesh(core_axis_name="c", subcore_axis_name="s")

    @functools.partial(
        pl.kernel,
        out_type=jax.ShapeDtypeStruct((B0, S, D), jnp.float32),
        mesh=mesh,
        scratch_types=[
            pltpu.VMEM((b_per_w,), jnp.int32),
            pltpu.VMEM((NBUF * CH, D), jnp.float32),
            pltpu.SemaphoreType.DMA((NBUF,)),
            pltpu.SemaphoreType.DMA((NBUF,)),
            pltpu.SemaphoreType.DMA((SLOTS,)),
            pltpu.VMEM_SHARED((16, SLOTS, CH, D), jnp.float32),
        ],
    )
    def emb(table_hbm, idx_hbm, out_hbm, idx_v, bufs, gsems, osems, dsems, shbuf):
        wid = lax.axis_index("s") * 2 + lax.axis_index("c")
        sid = lax.axis_index("s")
        row = wid // w_per_row
        col = (wid % w_per_row) * b_per_w
        pltpu.sync_copy(idx_hbm.at[row, pl.ds(col, b_per_w)], idx_v)

        def gather_start(c, j):
            pltpu.async_copy(
                table_hbm.at[idx_v.at[pl.ds(c * CH, CH)]],
                bufs.at[pl.ds(j * CH, CH)],
                gsems.at[j],
            )

        def gather_wait(j):
            pltpu.make_async_copy(
                table_hbm.at[pl.ds(0, CH)],
                bufs.at[pl.ds(j * CH, CH)],
                gsems.at[j],
            ).wait()

        def hop1_start(c, j):
            pltpu.async_copy(
                bufs.at[pl.ds(j * CH, CH)],
                shbuf.at[sid, lax.rem(c, SLOTS)],
                osems.at[j],
            )

        def hop1_wait(c):
            pltpu.make_async_copy(
                bufs.at[pl.ds(0, CH)],
                shbuf.at[0, 0],
                osems.at[lax.rem(c, NBUF)],
            ).wait()

        def drain_start(c):
            hop1_wait(c)
            pltpu.async_copy(
                shbuf.at[sid, lax.rem(c, SLOTS)],
                out_hbm.at[row, pl.ds(col + c * CH, CH)],
                dsems.at[lax.rem(c, SLOTS)],
            )

        def drain_wait(c):
            pltpu.make_async_copy(
                shbuf.at[0, 0],
                out_hbm.at[0, pl.ds(0, CH)],
                dsems.at[lax.rem(c, SLOTS)],
            ).wait()

        for c0 in range(LOOKAHEAD):
            gather_start(c0, c0)

        def body(c, _):
            j = lax.rem(c, NBUF)
            cg = c + LOOKAHEAD

            @pl.when(cg < n_ch)
            def _():
                gather_start(cg, lax.rem(cg, NBUF))

            gather_wait(j)

            rbase = j * CH

            @plsc.parallel_loop(0, CH)
            def _(r):
                for k in range(D // LANES):
                    sl = pl.ds(k * LANES, LANES)
                    bufs[rbase + r, sl] = bufs[rbase + r, sl] * SCALE

            @pl.when(c >= SLOTS)
            def _():
                drain_wait(c - SLOTS)

            hop1_start(c, j)

            @pl.when(c >= 1)
            def _():
                drain_start(c - 1)

            return 0

        lax.fori_loop(0, n_ch, body, 0)

        drain_start(n_ch - 1)
        for t in range(SLOTS):
            drain_wait(n_ch - SLOTS + t)

    return emb(table, x)


# speedup vs baseline: 1.0852x; 1.0852x over previous
import functools
import math

import jax
import jax.numpy as jnp
from jax import lax
from jax.experimental import pallas as pl
from jax.experimental.pallas import tpu as pltpu
from jax.experimental.pallas import tpu_sc as plsc

D_MODEL = 1024
SCALE = math.sqrt(D_MODEL)
LANES = 16
NW = 32
CH = 16
NBUF = 7
LOOKAHEAD = 5


def kernel(x, table):
    B0, S = x.shape
    V, D = table.shape
    x = x.astype(jnp.int32)
    b_per_w = (B0 * S) // NW
    w_per_row = S // b_per_w
    n_ch = b_per_w // CH

    mesh = plsc.VectorSubcoreMesh(core_axis_name="c", subcore_axis_name="s")

    @functools.partial(
        pl.kernel,
        out_type=jax.ShapeDtypeStruct((B0, S, D), jnp.float32),
        mesh=mesh,
        scratch_types=[
            pltpu.VMEM((b_per_w,), jnp.int32),
            pltpu.VMEM((NBUF * CH, D), jnp.float32),
            pltpu.SemaphoreType.DMA((NBUF,)),
            pltpu.SemaphoreType.DMA((NBUF,)),
        ],
    )
    def emb(table_hbm, idx_hbm, out_hbm, idx_v, bufs, gsems, osems):
        wid = lax.axis_index("s") * 2 + lax.axis_index("c")
        row = wid // w_per_row
        col = (wid % w_per_row) * b_per_w
        pltpu.sync_copy(idx_hbm.at[row, pl.ds(col, b_per_w)], idx_v)

        def gather_start(c, j):
            pltpu.async_copy(
                table_hbm.at[idx_v.at[pl.ds(c * CH, CH)]],
                bufs.at[pl.ds(j * CH, CH)],
                gsems.at[j],
            )

        def gather_wait(j):
            pltpu.make_async_copy(
                table_hbm.at[pl.ds(0, CH)],
                bufs.at[pl.ds(j * CH, CH)],
                gsems.at[j],
            ).wait()

        def wb_start(c, j):
            pltpu.async_copy(
                bufs.at[pl.ds(j * CH, CH)],
                out_hbm.at[row, pl.ds(col + c * CH, CH)],
                osems.at[j],
            )

        def wb_wait(j):
            pltpu.make_async_copy(
                bufs.at[pl.ds(j * CH, CH)],
                out_hbm.at[0, pl.ds(0, CH)],
                osems.at[j],
            ).wait()

        for c0 in range(LOOKAHEAD):
            gather_start(c0, c0)

        def body(c, _):
            j = lax.rem(c, NBUF)
            cg = c + LOOKAHEAD

            @pl.when(cg < n_ch)
            def _():
                jg = lax.rem(cg, NBUF)

                @pl.when(cg >= NBUF)
                def _():
                    wb_wait(jg)

                gather_start(cg, jg)

            gather_wait(j)

            rbase = j * CH

            @plsc.parallel_loop(0, CH)
            def _(r):
                for k in range(D // LANES):
                    sl = pl.ds(k * LANES, LANES)
                    bufs[rbase + r, sl] = bufs[rbase + r, sl] * SCALE

            wb_start(c, j)
            return 0

        lax.fori_loop(0, n_ch, body, 0)

        for jj in range(NBUF):
            wb_wait(jj)

    return emb(table, x)
